# baseline (speedup 1.0000x reference)
import functools

import jax
import jax.numpy as jnp
from jax import lax
from jax.experimental import pallas as pl
from jax.experimental.pallas import tpu as pltpu

Q = 1024
K = 100000
NN = 64
SUB = 8
SENT = 0x7FFFFFFF


def _radix_select(key, kk, alive, nbits):
    def cond(carry):
        i, _, kk, alive = carry
        return (i < nbits) & (alive != kk)

    def body(carry):
        i, key, kk, alive = carry
        b = nbits - 1 - i
        bitv = (key >> b) & 1
        cnt0 = jnp.sum(1 - bitv)
        take1 = kk > cnt0
        kk = jnp.where(take1, kk - cnt0, kk)
        alive = jnp.where(take1, alive - cnt0, cnt0)
        keep = jnp.where(take1, 1, 0)
        key = jnp.where(bitv == keep, key, SENT)
        return i + 1, key, kk, alive

    _, key, kk, alive = lax.while_loop(
        cond, body, (jnp.int32(0), key, kk, alive))
    return key, kk, alive


def _impute_block(dist_ref, fit_ref, mask_ref, out_ref):
    LK = K // SUB
    d = dist_ref[0]
    bits = lax.bitcast_convert_type(d, jnp.int32)

    CW = 128 if LK >= 16 * 128 else max(1, LK // 8)
    mins = []
    for c in range(0, LK, CW):
        mins.append(jnp.min(d[:, c:min(c + CW, LK)], axis=1, keepdims=True))
    cmin = jnp.concatenate(mins, axis=1)
    cbits = lax.bitcast_convert_type(cmin, jnp.int32)
    nch = cbits.size
    ckey, _, _ = _radix_select(
        cbits, jnp.int32(NN), jnp.int32(nch), 31)
    cact = ckey != SENT
    mb = jnp.max(jnp.where(cact, ckey, 0))
    L = jnp.min(cmin)
    Mf = lax.bitcast_convert_type(mb, jnp.float32)

    cand = bits <= mb
    scale = (2.0 ** 24) / jnp.maximum(Mf - L, 1e-30)
    q = ((jnp.minimum(d, Mf) - L) * scale).astype(jnp.int32)
    q0 = jnp.where(cand, q, SENT)
    alive0 = jnp.sum(jnp.where(cand, 1, 0))
    key, kk, alive = _radix_select(q0, jnp.int32(NN), alive0, 25)
    actq = key != SENT
    tq = jnp.min(key)

    keyb = jnp.where(actq, bits, SENT)
    keyb, kk, alive = _radix_select(keyb, kk, alive, 31)
    actb = keyb != SENT
    tb = jnp.min(keyb)

    idx = (lax.broadcasted_iota(jnp.int32, (SUB, LK), 0) * LK
           + lax.broadcasted_iota(jnp.int32, (SUB, LK), 1))
    key2 = jnp.where(actb, idx, SENT)
    key2, _, _ = _radix_select(key2, kk, alive, max(1, (K - 1).bit_length()))
    t2 = jnp.min(key2)

    sel = ((q0 < tq) | (actq & (bits < tb)) | (actb & (idx < t2))
           | (key2 != SENT))

    w = (1 - mask_ref[0]).astype(jnp.float32)
    fit = fit_ref[0]
    zero = jnp.zeros((), jnp.float32)
    sum_w = jnp.sum(jnp.where(sel, w, zero))
    sum_wx = jnp.sum(jnp.where(sel, w * fit, zero))
    div = jnp.where(sum_w == 0.0, 1.0, sum_w)
    out_ref[...] = (sum_wx / div).reshape(1, 1, 1)


@jax.jit
def _impute(dist3, fit3, mask3):
    LK = K // SUB
    out = pl.pallas_call(
        _impute_block,
        grid=(Q,),
        in_specs=[
            pl.BlockSpec((1, SUB, LK), lambda g: (g, 0, 0)),
            pl.BlockSpec((1, SUB, LK), lambda g: (0, 0, 0)),
            pl.BlockSpec((1, SUB, LK), lambda g: (0, 0, 0)),
        ],
        out_specs=pl.BlockSpec((1, 1, 1), lambda g: (g, 0, 0)),
        out_shape=jax.ShapeDtypeStruct((Q, 1, 1), jnp.float32),
        compiler_params=pltpu.CompilerParams(
            dimension_semantics=("arbitrary",),
        ),
    )(dist3, fit3, mask3)
    return out.reshape(Q)


def kernel(dist_pot_donors, n_neighbors, fit_X_col, mask_fit_X_col):
    del n_neighbors
    dist3 = dist_pot_donors.reshape(Q, SUB, K // SUB)
    fit3 = fit_X_col.reshape(1, SUB, K // SUB)
    mask3 = mask_fit_X_col.reshape(1, SUB, K // SUB)
    return _impute(dist3, fit3, mask3)


# speedup vs baseline: 2.0965x; 2.0965x over previous
import jax
import jax.numpy as jnp
from jax import lax
from jax.experimental import pallas as pl
from jax.experimental.pallas import tpu as pltpu

Q = 1024
K = 100000
NN = 64
ROWS = 8
SENT = 0x7FFFFFFF


def _prefix_select(vals, act, kk, alive, nbits):
    def cond(carry):
        i, _, kk, alive = carry
        return (i < nbits) & jnp.any(alive != kk)

    def body(carry):
        i, tp, kk, alive = carry
        b = nbits - 1 - i
        pred = (vals >> b) == (tp >> b)
        if act is not None:
            pred = pred & act
        cnt0 = jnp.sum(pred, axis=1, keepdims=True)
        take1 = kk > cnt0
        kk = jnp.where(take1, kk - cnt0, kk)
        alive = jnp.where(take1, alive - cnt0, cnt0)
        tp = tp | jnp.where(take1, 1 << b, 0)
        return i + 1, tp, kk, alive

    tp0 = jnp.zeros((vals.shape[0], 1), jnp.int32)
    i_end, tp, kk, alive = lax.while_loop(
        cond, body, (jnp.int32(0), tp0, kk, alive))
    b_done = nbits - i_end
    act_out = (vals >> b_done) == (tp >> b_done)
    low_out = vals < tp
    if act is not None:
        act_out = act_out & act
        low_out = low_out & act
    return act_out, low_out, kk, alive


def _impute_block(dist_ref, fit_ref, mask_ref, out_ref):
    d = dist_ref[...]
    bits = lax.bitcast_convert_type(d, jnp.int32)
    kk0 = jnp.full((ROWS, 1), NN, dtype=jnp.int32)

    CH = 1024 if K >= 128 * 1024 // 2 else max(1, K // 128)
    mins = []
    for c in range(0, K, CH):
        mins.append(jnp.min(d[:, c:min(c + CH, K)], axis=1, keepdims=True))
    cmin = jnp.concatenate(mins, axis=1)
    nch = cmin.shape[1]
    cbits = lax.bitcast_convert_type(cmin, jnp.int32)
    cact, _, _, _ = _prefix_select(
        cbits, None, kk0, jnp.full((ROWS, 1), nch, jnp.int32), 31)
    mb = jnp.max(jnp.where(cact, cbits, 0), axis=1, keepdims=True)
    L = jnp.min(cmin, axis=1, keepdims=True)
    Mf = lax.bitcast_convert_type(mb, jnp.float32)

    cand = bits <= mb
    scale = (2.0 ** 24) / jnp.maximum(Mf - L, 1e-30)
    q = ((jnp.minimum(d, Mf) - L) * scale).astype(jnp.int32)
    q0 = jnp.where(cand, q, SENT)
    alive0 = jnp.sum(jnp.where(cand, 1, 0), axis=1, keepdims=True)
    act1, low1, kk, alive = _prefix_select(q0, None, kk0, alive0, 25)

    act2, low2, kk, alive = _prefix_select(bits, act1, kk, alive, 31)

    idx = lax.broadcasted_iota(jnp.int32, (ROWS, K), 1)
    act3, low3, _, _ = _prefix_select(
        idx, act2, kk, alive, max(1, (K - 1).bit_length()))

    sel = low1 | low2 | low3 | act3

    w = (1 - mask_ref[...]).astype(jnp.float32)
    fit = fit_ref[...]
    zero = jnp.zeros((), jnp.float32)
    sum_w = jnp.sum(jnp.where(sel, w, zero), axis=1, keepdims=True)
    sum_wx = jnp.sum(jnp.where(sel, w * fit, zero), axis=1, keepdims=True)
    div = jnp.where(sum_w == 0.0, 1.0, sum_w)
    out_ref[...] = sum_wx / div


@jax.jit
def _impute(dist, fit2d, mask2d):
    out = pl.pallas_call(
        _impute_block,
        grid=(Q // ROWS,),
        in_specs=[
            pl.BlockSpec((ROWS, K), lambda g: (g, 0)),
            pl.BlockSpec((1, K), lambda g: (0, 0)),
            pl.BlockSpec((1, K), lambda g: (0, 0)),
        ],
        out_specs=pl.BlockSpec((ROWS, 1), lambda g: (g, 0)),
        out_shape=jax.ShapeDtypeStruct((Q, 1), jnp.float32),
        compiler_params=pltpu.CompilerParams(
            dimension_semantics=("arbitrary",),
        ),
    )(dist, fit2d, mask2d)
    return jnp.squeeze(out, axis=1)


def kernel(dist_pot_donors, n_neighbors, fit_X_col, mask_fit_X_col):
    del n_neighbors
    fit2d = fit_X_col.reshape(1, K)
    mask2d = mask_fit_X_col.reshape(1, K)
    return _impute(dist_pot_donors, fit2d, mask2d)


# speedup vs baseline: 2.4951x; 1.1901x over previous
import jax
import jax.numpy as jnp
from jax import lax
from jax.experimental import pallas as pl
from jax.experimental.pallas import tpu as pltpu

Q = 1024
K = 100000
NN = 64
ROWS = 8
SENT = 0x7FFFFFFF


def _prefix_select(vals, act, kk, alive, nbits, enable=None):
    def cond(carry):
        i, _, kk, alive = carry
        go = (i < nbits) & jnp.any(alive != kk)
        if enable is not None:
            go = go & enable
        return go

    def body(carry):
        i, tp, kk, alive = carry
        b = nbits - 1 - i
        pred = (vals >> b) == (tp >> b)
        if act is not None:
            pred = pred & act
        cnt0 = jnp.sum(pred, axis=1, keepdims=True)
        take1 = kk > cnt0
        kk = jnp.where(take1, kk - cnt0, kk)
        alive = jnp.where(take1, alive - cnt0, cnt0)
        tp = tp | jnp.where(take1, 1 << b, 0)
        return i + 1, tp, kk, alive

    tp0 = jnp.zeros((vals.shape[0], 1), jnp.int32)
    i_end, tp, kk, alive = lax.while_loop(
        cond, body, (jnp.int32(0), tp0, kk, alive))
    b_done = nbits - i_end
    act_out = (vals >> b_done) == (tp >> b_done)
    low_out = vals < tp
    if act is not None:
        act_out = act_out & act
        low_out = low_out & act
    return act_out, low_out, kk, alive


def _impute_block(dist_ref, fit_ref, mask_ref, out_ref):
    d = dist_ref[...]
    bits = lax.bitcast_convert_type(d, jnp.int32)
    kk0 = jnp.full((ROWS, 1), NN, dtype=jnp.int32)

    CH = 1024 if K >= 128 * 1024 // 2 else max(1, K // 128)
    mins = []
    for c in range(0, K, CH):
        mins.append(jnp.min(d[:, c:min(c + CH, K)], axis=1, keepdims=True))
    cmin = jnp.concatenate(mins, axis=1)
    nch = cmin.shape[1]
    cbits = lax.bitcast_convert_type(cmin, jnp.int32)
    cact, _, _, _ = _prefix_select(
        cbits, None, kk0, jnp.full((ROWS, 1), nch, jnp.int32), 31)
    mb = jnp.max(jnp.where(cact, cbits, 0), axis=1, keepdims=True)
    L = jnp.min(cmin, axis=1, keepdims=True)
    Mf = lax.bitcast_convert_type(mb, jnp.float32)

    cand = bits <= mb
    scale = (2.0 ** 24) / jnp.maximum(Mf - L, 1e-30)
    q = ((jnp.minimum(d, Mf) - L) * scale).astype(jnp.int32)
    q0 = jnp.where(cand, q, SENT)
    alive0 = jnp.sum(jnp.where(cand, 1, 0), axis=1, keepdims=True)
    act1, low1, kk, alive = _prefix_select(q0, None, kk0, alive0, 25)

    w = (1 - mask_ref[...]).astype(jnp.float32)
    fit = fit_ref[...]
    zero = jnp.zeros((), jnp.float32)

    def _sums(sel):
        sum_w = jnp.sum(jnp.where(sel, w, zero), axis=1, keepdims=True)
        sum_wx = jnp.sum(jnp.where(sel, w * fit, zero), axis=1, keepdims=True)
        return sum_w, sum_wx

    def fast_path(_):
        return _sums(low1 | act1)

    def slow_path(_):
        tmin = jnp.min(jnp.where(act1, bits, SENT), axis=1, keepdims=True)
        tmax = jnp.max(jnp.where(act1, bits, 0), axis=1, keepdims=True)
        need2 = jnp.any((alive != kk) & (tmin != tmax))
        act2, low2, kk2, alive2 = _prefix_select(
            bits, act1, kk, alive, 31, enable=need2)

        idx = lax.broadcasted_iota(jnp.int32, (ROWS, K), 1)
        act3, low3, _, _ = _prefix_select(
            idx, act2, kk2, alive2, max(1, (K - 1).bit_length()))
        return _sums(low1 | low2 | low3 | act3)

    sum_w, sum_wx = lax.cond(jnp.all(alive == kk), fast_path, slow_path, 0)
    div = jnp.where(sum_w == 0.0, 1.0, sum_w)
    out_ref[...] = sum_wx / div


@jax.jit
def _impute(dist, fit2d, mask2d):
    out = pl.pallas_call(
        _impute_block,
        grid=(Q // ROWS,),
        in_specs=[
            pl.BlockSpec((ROWS, K), lambda g: (g, 0)),
            pl.BlockSpec((1, K), lambda g: (0, 0)),
            pl.BlockSpec((1, K), lambda g: (0, 0)),
        ],
        out_specs=pl.BlockSpec((ROWS, 1), lambda g: (g, 0)),
        out_shape=jax.ShapeDtypeStruct((Q, 1), jnp.float32),
        compiler_params=pltpu.CompilerParams(
            dimension_semantics=("arbitrary",),
        ),
    )(dist, fit2d, mask2d)
    return jnp.squeeze(out, axis=1)


def kernel(dist_pot_donors, n_neighbors, fit_X_col, mask_fit_X_col):
    del n_neighbors
    fit2d = fit_X_col.reshape(1, K)
    mask2d = mask_fit_X_col.reshape(1, K)
    return _impute(dist_pot_donors, fit2d, mask2d)


# speedup vs baseline: 2.5240x; 1.0116x over previous
import jax
import jax.numpy as jnp
from jax import lax
from jax.experimental import pallas as pl
from jax.experimental.pallas import tpu as pltpu

Q = 1024
K = 100000
NN = 64
ROWS = 8
SENT = 0x7FFFFFFF


def _prefix_select(vals, act, kk, alive, nbits, enable=None):
    def cond(carry):
        i, _, kk, alive = carry
        go = (i < nbits) & jnp.any(alive != kk)
        if enable is not None:
            go = go & enable
        return go

    def body(carry):
        i, tp, kk, alive = carry
        b = nbits - 1 - i
        pred = (vals >> b) == (tp >> b)
        if act is not None:
            pred = pred & act
        cnt0 = jnp.sum(pred, axis=1, keepdims=True)
        take1 = kk > cnt0
        kk = jnp.where(take1, kk - cnt0, kk)
        alive = jnp.where(take1, alive - cnt0, cnt0)
        tp = tp | jnp.where(take1, 1 << b, 0)
        return i + 1, tp, kk, alive

    tp0 = jnp.zeros((vals.shape[0], 1), jnp.int32)
    i_end, tp, kk, alive = lax.while_loop(
        cond, body, (jnp.int32(0), tp0, kk, alive))
    b_done = nbits - i_end
    act_out = (vals >> b_done) == (tp >> b_done)
    low_out = vals < tp
    if act is not None:
        act_out = act_out & act
        low_out = low_out & act
    return act_out, low_out, kk, alive


def _impute_block(dist_ref, fit_ref, mask_ref, out_ref):
    d = dist_ref[...]
    bits = lax.bitcast_convert_type(d, jnp.int32)
    kk0 = jnp.full((ROWS, 1), NN, dtype=jnp.int32)

    CH = 512 if K >= 128 * 512 else max(1, K // 128)
    mins = []
    for c in range(0, K, CH):
        mins.append(jnp.min(d[:, c:min(c + CH, K)], axis=1, keepdims=True))
    cmin = jnp.concatenate(mins, axis=1)
    nch = cmin.shape[1]
    cbits = lax.bitcast_convert_type(cmin, jnp.int32)
    cact, _, _, _ = _prefix_select(
        cbits, None, kk0, jnp.full((ROWS, 1), nch, jnp.int32), 31)
    mb = jnp.max(jnp.where(cact, cbits, 0), axis=1, keepdims=True)
    L = jnp.min(cmin, axis=1, keepdims=True)
    Mf = lax.bitcast_convert_type(mb, jnp.float32)

    cand = bits <= mb
    scale = (2.0 ** 24) / jnp.maximum(Mf - L, 1e-30)
    q = ((d - L) * scale).astype(jnp.int32)
    q0 = jnp.where(cand, q, SENT)
    alive0 = jnp.sum(jnp.where(cand, 1, 0), axis=1, keepdims=True)
    act1, low1, kk, alive = _prefix_select(q0, None, kk0, alive0, 25)

    w = (1 - mask_ref[...]).astype(jnp.float32)
    fit = fit_ref[...]
    zero = jnp.zeros((), jnp.float32)

    def _sums(sel):
        sum_w = jnp.sum(jnp.where(sel, w, zero), axis=1, keepdims=True)
        sum_wx = jnp.sum(jnp.where(sel, w * fit, zero), axis=1, keepdims=True)
        return sum_w, sum_wx

    def fast_path(_):
        return _sums(low1 | act1)

    def slow_path(_):
        tmin = jnp.min(jnp.where(act1, bits, SENT), axis=1, keepdims=True)
        tmax = jnp.max(jnp.where(act1, bits, 0), axis=1, keepdims=True)
        need2 = jnp.any((alive != kk) & (tmin != tmax))
        act2, low2, kk2, alive2 = _prefix_select(
            bits, act1, kk, alive, 31, enable=need2)

        idx = lax.broadcasted_iota(jnp.int32, (ROWS, K), 1)
        act3, low3, _, _ = _prefix_select(
            idx, act2, kk2, alive2, max(1, (K - 1).bit_length()))
        return _sums(low1 | low2 | low3 | act3)

    sum_w, sum_wx = lax.cond(jnp.all(alive == kk), fast_path, slow_path, 0)
    div = jnp.where(sum_w == 0.0, 1.0, sum_w)
    out_ref[...] = sum_wx / div


@jax.jit
def _impute(dist, fit2d, mask2d):
    out = pl.pallas_call(
        _impute_block,
        grid=(Q // ROWS,),
        in_specs=[
            pl.BlockSpec((ROWS, K), lambda g: (g, 0)),
            pl.BlockSpec((1, K), lambda g: (0, 0)),
            pl.BlockSpec((1, K), lambda g: (0, 0)),
        ],
        out_specs=pl.BlockSpec((ROWS, 1), lambda g: (g, 0)),
        out_shape=jax.ShapeDtypeStruct((Q, 1), jnp.float32),
        compiler_params=pltpu.CompilerParams(
            dimension_semantics=("arbitrary",),
        ),
    )(dist, fit2d, mask2d)
    return jnp.squeeze(out, axis=1)


def kernel(dist_pot_donors, n_neighbors, fit_X_col, mask_fit_X_col):
    del n_neighbors
    fit2d = fit_X_col.reshape(1, K)
    mask2d = mask_fit_X_col.reshape(1, K)
    return _impute(dist_pot_donors, fit2d, mask2d)


# speedup vs baseline: 3.5108x; 1.3910x over previous
import jax
import jax.numpy as jnp
from jax import lax
from jax.experimental import pallas as pl
from jax.experimental.pallas import tpu as pltpu

Q = 1024
K = 100000
NN = 64
ROWS = 16
SENT = 0x7FFFFFFF


def _prefix_select(vals, act, kk, alive, nbits, enable=None):
    def cond(carry):
        i, _, kk, alive = carry
        go = (i < nbits) & jnp.any(alive != kk)
        if enable is not None:
            go = go & enable
        return go

    def body(carry):
        i, tp, kk, alive = carry
        b = nbits - 1 - i
        pred = (vals >> b) == (tp >> b)
        if act is not None:
            pred = pred & act
        cnt0 = jnp.sum(pred, axis=1, keepdims=True)
        take1 = kk > cnt0
        kk = jnp.where(take1, kk - cnt0, kk)
        alive = jnp.where(take1, alive - cnt0, cnt0)
        tp = tp | jnp.where(take1, 1 << b, 0)
        return i + 1, tp, kk, alive

    tp0 = jnp.zeros((vals.shape[0], 1), jnp.int32)
    i_end, tp, kk, alive = lax.while_loop(
        cond, body, (jnp.int32(0), tp0, kk, alive))
    b_done = nbits - i_end
    act_out = (vals >> b_done) == (tp >> b_done)
    low_out = vals < tp
    if act is not None:
        act_out = act_out & act
        low_out = low_out & act
    return act_out, low_out, kk, alive


def _impute_block(dist_ref, fit_ref, mask_ref, out_ref):
    d = dist_ref[...]
    bits = lax.bitcast_convert_type(d, jnp.int32)
    kk0 = jnp.full((ROWS, 1), NN, dtype=jnp.int32)

    CH = 512 if K >= 128 * 512 else max(1, K // 128)
    mins = []
    for c in range(0, K, CH):
        mins.append(jnp.min(d[:, c:min(c + CH, K)], axis=1, keepdims=True))
    cmin = jnp.concatenate(mins, axis=1)
    nch = cmin.shape[1]
    cbits = lax.bitcast_convert_type(cmin, jnp.int32)
    cact, _, _, _ = _prefix_select(
        cbits, None, kk0, jnp.full((ROWS, 1), nch, jnp.int32), 31)
    mb = jnp.max(jnp.where(cact, cbits, 0), axis=1, keepdims=True)
    L = jnp.min(cmin, axis=1, keepdims=True)
    Mf = lax.bitcast_convert_type(mb, jnp.float32)

    cand = bits <= mb
    scale = (2.0 ** 24) / jnp.maximum(Mf - L, 1e-30)
    q = ((d - L) * scale).astype(jnp.int32)
    q0 = jnp.where(cand, q, SENT)
    alive0 = jnp.sum(jnp.where(cand, 1, 0), axis=1, keepdims=True)
    act1, low1, kk, alive = _prefix_select(q0, None, kk0, alive0, 25)

    w = (1 - mask_ref[...]).astype(jnp.float32)
    fit = fit_ref[...]
    zero = jnp.zeros((), jnp.float32)

    def _sums(sel):
        sum_w = jnp.sum(jnp.where(sel, w, zero), axis=1, keepdims=True)
        sum_wx = jnp.sum(jnp.where(sel, w * fit, zero), axis=1, keepdims=True)
        return sum_w, sum_wx

    def fast_path(_):
        return _sums(low1 | act1)

    def slow_path(_):
        tmin = jnp.min(jnp.where(act1, bits, SENT), axis=1, keepdims=True)
        tmax = jnp.max(jnp.where(act1, bits, 0), axis=1, keepdims=True)
        need2 = jnp.any((alive != kk) & (tmin != tmax))
        act2, low2, kk2, alive2 = _prefix_select(
            bits, act1, kk, alive, 31, enable=need2)

        idx = lax.broadcasted_iota(jnp.int32, (ROWS, K), 1)
        act3, low3, _, _ = _prefix_select(
            idx, act2, kk2, alive2, max(1, (K - 1).bit_length()))
        return _sums(low1 | low2 | low3 | act3)

    sum_w, sum_wx = lax.cond(jnp.all(alive == kk), fast_path, slow_path, 0)
    div = jnp.where(sum_w == 0.0, 1.0, sum_w)
    out_ref[...] = sum_wx / div


@jax.jit
def _impute(dist, fit2d, mask2d):
    out = pl.pallas_call(
        _impute_block,
        grid=(Q // ROWS,),
        in_specs=[
            pl.BlockSpec((ROWS, K), lambda g: (g, 0)),
            pl.BlockSpec((1, K), lambda g: (0, 0)),
            pl.BlockSpec((1, K), lambda g: (0, 0)),
        ],
        out_specs=pl.BlockSpec((ROWS, 1), lambda g: (g, 0)),
        out_shape=jax.ShapeDtypeStruct((Q, 1), jnp.float32),
        compiler_params=pltpu.CompilerParams(
            dimension_semantics=("arbitrary",),
        ),
    )(dist, fit2d, mask2d)
    return jnp.squeeze(out, axis=1)


def kernel(dist_pot_donors, n_neighbors, fit_X_col, mask_fit_X_col):
    del n_neighbors
    fit2d = fit_X_col.reshape(1, K)
    mask2d = mask_fit_X_col.reshape(1, K)
    return _impute(dist_pot_donors, fit2d, mask2d)


# speedup vs baseline: 3.5432x; 1.0092x over previous
import jax
import jax.numpy as jnp
from jax import lax
from jax.experimental import pallas as pl
from jax.experimental.pallas import tpu as pltpu

Q = 1024
K = 100000
NN = 64
ROWS = 16
SENT = 0x7FFFFFFF


def _prefix_select(vals, act, kk, alive, nbits, enable=None):
    def cond(carry):
        i, _, kk, alive = carry
        go = (i < nbits) & jnp.any(alive != kk)
        if enable is not None:
            go = go & enable
        return go

    def body(carry):
        i, tp, kk, alive = carry
        b = nbits - 1 - i
        pred = (vals >> b) == (tp >> b)
        if act is not None:
            pred = pred & act
        cnt0 = jnp.sum(pred, axis=1, keepdims=True)
        take1 = kk > cnt0
        kk = jnp.where(take1, kk - cnt0, kk)
        alive = jnp.where(take1, alive - cnt0, cnt0)
        tp = tp | jnp.where(take1, 1 << b, 0)
        return i + 1, tp, kk, alive

    tp0 = jnp.zeros((vals.shape[0], 1), jnp.int32)
    i_end, tp, kk, alive = lax.while_loop(
        cond, body, (jnp.int32(0), tp0, kk, alive))
    b_done = nbits - i_end
    act_out = (vals >> b_done) == (tp >> b_done)
    low_out = vals < tp
    if act is not None:
        act_out = act_out & act
        low_out = low_out & act
    return act_out, low_out, kk, alive


def _impute_block(dist_ref, fit_ref, mask_ref, out_ref):
    d = dist_ref[...]
    kk0 = jnp.full((ROWS, 1), NN, dtype=jnp.int32)

    CH = 512 if K >= 128 * 512 else max(1, K // 128)
    mins = []
    for c in range(0, K, CH):
        mins.append(jnp.min(d[:, c:min(c + CH, K)], axis=1, keepdims=True))
    cmin = jnp.concatenate(mins, axis=1)
    nch = cmin.shape[1]
    cbits = lax.bitcast_convert_type(cmin, jnp.int32)
    cact, _, _, _ = _prefix_select(
        cbits, None, kk0, jnp.full((ROWS, 1), nch, jnp.int32), 31)
    mb = jnp.max(jnp.where(cact, cbits, 0), axis=1, keepdims=True)
    L = jnp.min(cmin, axis=1, keepdims=True)
    Mf = lax.bitcast_convert_type(mb, jnp.float32)

    cand = d <= Mf
    scale = (2.0 ** 24) / jnp.maximum(Mf - L, 1e-30)
    q = ((d - L) * scale).astype(jnp.int32)
    q0 = jnp.where(cand, q, SENT)
    alive0 = jnp.sum(jnp.where(cand, 1, 0), axis=1, keepdims=True)
    act1, low1, kk, alive = _prefix_select(q0, None, kk0, alive0, 25)

    w = (1 - mask_ref[...]).astype(jnp.float32)
    fit = fit_ref[...]
    zero = jnp.zeros((), jnp.float32)

    def _sums(sel):
        sum_w = jnp.sum(jnp.where(sel, w, zero), axis=1, keepdims=True)
        sum_wx = jnp.sum(jnp.where(sel, w * fit, zero), axis=1, keepdims=True)
        return sum_w, sum_wx

    def fast_path(_):
        return _sums(low1 | act1)

    def slow_path(_):
        bits = lax.bitcast_convert_type(dist_ref[...], jnp.int32)
        tmin = jnp.min(jnp.where(act1, bits, SENT), axis=1, keepdims=True)
        tmax = jnp.max(jnp.where(act1, bits, 0), axis=1, keepdims=True)
        need2 = jnp.any((alive != kk) & (tmin != tmax))
        act2, low2, kk2, alive2 = _prefix_select(
            bits, act1, kk, alive, 31, enable=need2)

        idx = lax.broadcasted_iota(jnp.int32, (ROWS, K), 1)
        act3, low3, _, _ = _prefix_select(
            idx, act2, kk2, alive2, max(1, (K - 1).bit_length()))
        return _sums(low1 | low2 | low3 | act3)

    sum_w, sum_wx = lax.cond(jnp.all(alive == kk), fast_path, slow_path, 0)
    div = jnp.where(sum_w == 0.0, 1.0, sum_w)
    out_ref[...] = sum_wx / div


@jax.jit
def _impute(dist, fit2d, mask2d):
    out = pl.pallas_call(
        _impute_block,
        grid=(Q // ROWS,),
        in_specs=[
            pl.BlockSpec((ROWS, K), lambda g: (g, 0)),
            pl.BlockSpec((1, K), lambda g: (0, 0)),
            pl.BlockSpec((1, K), lambda g: (0, 0)),
        ],
        out_specs=pl.BlockSpec((ROWS, 1), lambda g: (g, 0)),
        out_shape=jax.ShapeDtypeStruct((Q, 1), jnp.float32),
        compiler_params=pltpu.CompilerParams(
            dimension_semantics=("arbitrary",),
        ),
    )(dist, fit2d, mask2d)
    return jnp.squeeze(out, axis=1)


def kernel(dist_pot_donors, n_neighbors, fit_X_col, mask_fit_X_col):
    del n_neighbors
    fit2d = fit_X_col.reshape(1, K)
    mask2d = mask_fit_X_col.reshape(1, K)
    return _impute(dist_pot_donors, fit2d, mask2d)
